# baseline (speedup 1.0000x reference)
import functools

import jax
import jax.numpy as jnp
from jax import lax
from jax.experimental import pallas as pl
from jax.experimental.pallas import tpu as pltpu
from jax.experimental.pallas import tpu_sc as plsc

E = 8
H = 128
N = 131072
NW = 32
C = N // NW
B = 1024
G = 144
NPAD = G * B

_mesh = plsc.VectorSubcoreMesh(core_axis_name="c", subcore_axis_name="s")


def _wid():
    return lax.axis_index("s") * 2 + lax.axis_index("c")


def _expert_id(px, py, pz):
    c0 = ((px + 1.0) >= 1.0).astype(jnp.int32)
    c1 = ((py + 1.0) >= 1.0).astype(jnp.int32)
    c2 = ((pz + 1.0) >= 1.0).astype(jnp.int32)
    return c2 + 2 * c1 + 4 * c0


@functools.partial(
    pl.kernel,
    out_type=(
        jax.ShapeDtypeStruct((NW, 16), jnp.int32),
        jax.ShapeDtypeStruct((NW, C), jnp.int32),
    ),
    mesh=_mesh,
    compiler_params=pltpu.CompilerParams(needs_layout_passes=False),
    scratch_types=(
        pltpu.VMEM((3 * C,), jnp.float32),
        pltpu.VMEM((C,), jnp.int32),
        pltpu.VMEM((16,), jnp.int32),
    ),
)
def _sc_route(xflat_hbm, counts_hbm, ev_hbm, xv, evv, cntv):
    w = _wid()
    pltpu.sync_copy(xflat_hbm.at[pl.ds(w * 3 * C, 3 * C)], xv)
    l16 = jnp.arange(16, dtype=jnp.int32)
    zero = jnp.zeros((16,), jnp.int32)

    def chunk(i, accs):
        idx3 = (i * 16 + l16) * 3
        px = plsc.load_gather(xv, [idx3])
        py = plsc.load_gather(xv, [idx3 + 1])
        pz = plsc.load_gather(xv, [idx3 + 2])
        ev = _expert_id(px, py, pz)
        evv[pl.ds(i * 16, 16)] = ev
        return tuple(accs[e] + (ev == e).astype(jnp.int32) for e in range(E))

    accs = lax.fori_loop(0, C // 16, chunk, (zero,) * E)
    cnt = zero
    for e in range(E):
        cnt = cnt + jnp.where(l16 == e, jnp.sum(accs[e]), 0)
    cntv[...] = cnt
    pltpu.sync_copy(evv, ev_hbm.at[w])
    pltpu.sync_copy(cntv, counts_hbm.at[w])


@functools.partial(
    pl.kernel,
    out_type=(
        jax.ShapeDtypeStruct((3 * NPAD,), jnp.float32),
        jax.ShapeDtypeStruct((N // 128, 128), jnp.int32),
        jax.ShapeDtypeStruct((G,), jnp.int32),
    ),
    mesh=_mesh,
    compiler_params=pltpu.CompilerParams(needs_layout_passes=False),
    scratch_types=(
        pltpu.VMEM((3 * C,), jnp.float32),
        pltpu.VMEM((C,), jnp.int32),
        pltpu.VMEM((NW, 16), jnp.int32),
        pltpu.VMEM((16,), jnp.int32),
        pltpu.VMEM((G,), jnp.int32),
        pltpu.VMEM((C // 128, 128), jnp.int32),
        pltpu.VMEM((3, C // 128, 128), jnp.int32),
        pltpu.VMEM((3, C // 128, 128), jnp.float32),
        pltpu.SemaphoreType.DMA,
    ),
)
def _sc_scatter(xflat_hbm, ev_hbm, counts_hbm, xs_hbm, d_hbm, b2e_hbm,
                xv, evv, cmv, baseref, b2ev, dref, icb, xcb, sem):
    w = _wid()
    pltpu.sync_copy(xflat_hbm.at[pl.ds(w * 3 * C, 3 * C)], xv)
    pltpu.sync_copy(ev_hbm.at[w], evv)
    pltpu.sync_copy(counts_hbm, cmv)
    l16 = jnp.arange(16, dtype=jnp.int32)
    zero = jnp.zeros((16,), jnp.int32)

    def accrow(j, st):
        tot, pre = st
        row = cmv[j]
        return tot + row, pre + jnp.where(j < w, row, 0)

    tot, pre = lax.fori_loop(0, NW, accrow, (zero, zero))
    cap = ((tot + (B - 1)) // B) * B
    padoff = plsc.cumsum(cap) - cap
    baseref[...] = padoff + pre

    @pl.when(w == 0)
    def _():
        ends = padoff + cap
        end_s = [jnp.sum(jnp.where(l16 == e, ends, 0)) for e in range(E)]
        for i in range(G // 16):
            gv = (i * 16 + l16) * B
            cntv = jnp.zeros((16,), jnp.int32)
            for e in range(E):
                cntv = cntv + (gv >= end_s[e]).astype(jnp.int32)
            b2ev[pl.ds(i * 16, 16)] = jnp.minimum(cntv, E - 1)
        pltpu.sync_copy(b2ev, b2e_hbm)

    def row_fn(r, _):
        for j in range(8):
            off = r * 128 + j * 16
            ev = evv[pl.ds(off, 16)]
            g = plsc.load_gather(baseref, [ev])
            rank = jnp.zeros((16,), jnp.int32)
            bnew = baseref[...]
            for e in range(E):
                m = ev == e
                mi = m.astype(jnp.int32)
                cs = plsc.cumsum(mi)
                rank = rank + jnp.where(m, cs - 1, 0)
                bnew = bnew + jnp.where(l16 == e, jnp.sum(mi), 0)
            baseref[...] = bnew
            d = g + rank
            dref[r, pl.ds(j * 16, 16)] = d
            for c in range(3):
                v = plsc.load_gather(xv, [(off + l16) * 3 + c])
                xcb[c, r, pl.ds(j * 16, 16)] = v
                icb[c, r, pl.ds(j * 16, 16)] = 3 * d + c
        return 0

    lax.fori_loop(0, C // 128, row_fn, 0)

    def fire(r, _):
        for c in range(3):
            pltpu.async_copy(xcb.at[c, r], xs_hbm.at[icb.at[c, r]], sem)
        return 0

    def drain(r, _):
        for c in range(3):
            pltpu.make_async_copy(xcb.at[c, r], xs_hbm.at[icb.at[c, r]], sem).wait()
        return 0

    lax.fori_loop(0, C // 128, fire, 0)
    lax.fori_loop(0, C // 128, drain, 0)
    pltpu.sync_copy(dref, d_hbm.at[pl.ds(w * (C // 128), C // 128), :])


@functools.partial(
    pl.kernel,
    out_type=jax.ShapeDtypeStruct((N // 128, 128), jnp.float32),
    mesh=_mesh,
    compiler_params=pltpu.CompilerParams(needs_layout_passes=False),
    scratch_types=(
        pltpu.VMEM((C // 128, 128), jnp.int32),
        pltpu.VMEM((C // 128, 128), jnp.float32),
        pltpu.SemaphoreType.DMA,
    ),
)
def _sc_gather_back(d_hbm, ys_hbm, y_hbm, dv, yv, sem):
    w = _wid()
    rows = C // 128
    pltpu.sync_copy(d_hbm.at[pl.ds(w * rows, rows), :], dv)

    def fire(r, _):
        pltpu.async_copy(ys_hbm.at[dv.at[r]], yv.at[r], sem)
        return 0

    def drain(r, _):
        pltpu.make_async_copy(ys_hbm.at[dv.at[r]], yv.at[r], sem).wait()
        return 0

    lax.fori_loop(0, rows, fire, 0)
    lax.fori_loop(0, rows, drain, 0)
    pltpu.sync_copy(yv, y_hbm.at[pl.ds(w * rows, rows), :])


def _mlp_block(b2e_ref, x_ref, W1_ref, b1_ref, W2_ref, b2_ref, W3_ref, b3_ref,
               o_ref):
    x0 = x_ref[:, 0:1]
    x1 = x_ref[:, 1:2]
    x2 = x_ref[:, 2:3]
    h1 = jax.nn.relu(
        x0 * W1_ref[0, 0:1, :] + x1 * W1_ref[0, 1:2, :] + x2 * W1_ref[0, 2:3, :]
        + b1_ref[0]
    )
    h2 = jax.nn.relu(
        jnp.dot(h1, W2_ref[0], preferred_element_type=jnp.float32) + b2_ref[0]
    )
    o_ref[...] = jnp.sum(h2 * W3_ref[0], axis=1, keepdims=True) + b3_ref[0]


def kernel(x, W1, b1, W2, b2, W3, b3):
    xflat = x.reshape(-1)
    counts, ev = _sc_route(xflat)
    xs_flat, d, b2e = _sc_scatter(xflat, ev, counts)
    xs = xs_flat.reshape(NPAD, 3)
    W3r = W3.reshape(E, 1, H)
    b1r = b1.reshape(E, 1, H)
    b2r = b2.reshape(E, 1, H)
    b3r = b3.reshape(E, 1, 1)

    grid_spec = pltpu.PrefetchScalarGridSpec(
        num_scalar_prefetch=1,
        grid=(G,),
        in_specs=[
            pl.BlockSpec((B, 3), lambda g, b2e_ref: (g, 0)),
            pl.BlockSpec((1, 3, H), lambda g, b2e_ref: (b2e_ref[g], 0, 0)),
            pl.BlockSpec((1, 1, H), lambda g, b2e_ref: (b2e_ref[g], 0, 0)),
            pl.BlockSpec((1, H, H), lambda g, b2e_ref: (b2e_ref[g], 0, 0)),
            pl.BlockSpec((1, 1, H), lambda g, b2e_ref: (b2e_ref[g], 0, 0)),
            pl.BlockSpec((1, 1, H), lambda g, b2e_ref: (b2e_ref[g], 0, 0)),
            pl.BlockSpec((1, 1, 1), lambda g, b2e_ref: (b2e_ref[g], 0, 0)),
        ],
        out_specs=pl.BlockSpec((B, 1), lambda g, b2e_ref: (g, 0)),
    )
    ys = pl.pallas_call(
        _mlp_block,
        grid_spec=grid_spec,
        out_shape=jax.ShapeDtypeStruct((NPAD, 1), jnp.float32),
    )(b2e, xs, W1, b1r, W2, b2r, W3r, b3r)

    y = _sc_gather_back(d, ys.reshape(-1))
    return y.reshape(N, 1)


# speedup vs baseline: 1.1008x; 1.1008x over previous
import functools

import jax
import jax.numpy as jnp
from jax import lax
from jax.experimental import pallas as pl
from jax.experimental.pallas import tpu as pltpu
from jax.experimental.pallas import tpu_sc as plsc

E = 8
H = 128
N = 131072
NW = 32
C = N // NW
B = 1024
G = 144
NPAD = G * B
ROWS = C // 128

_mesh = plsc.VectorSubcoreMesh(core_axis_name="c", subcore_axis_name="s")
_params = pltpu.CompilerParams(needs_layout_passes=False)


def _wid():
    return lax.axis_index("s") * 2 + lax.axis_index("c")


def _expert_id(px, py, pz):
    c0 = ((px + 1.0) >= 1.0).astype(jnp.int32)
    c1 = ((py + 1.0) >= 1.0).astype(jnp.int32)
    c2 = ((pz + 1.0) >= 1.0).astype(jnp.int32)
    return c2 + 2 * c1 + 4 * c0


@functools.partial(
    pl.kernel,
    out_type=(
        jax.ShapeDtypeStruct((NW, E * 16), jnp.int32),
        jax.ShapeDtypeStruct((NW, C), jnp.int32),
    ),
    mesh=_mesh,
    compiler_params=_params,
    scratch_types=(
        pltpu.VMEM((3 * C,), jnp.float32),
        pltpu.VMEM((C,), jnp.int32),
        pltpu.VMEM((E * 16,), jnp.int32),
    ),
)
def _sc_route(xflat_hbm, counts_hbm, ev_hbm, xv, evv, cntv):
    w = _wid()
    pltpu.sync_copy(xflat_hbm.at[pl.ds(w * 3 * C, 3 * C)], xv)
    l16 = jnp.arange(16, dtype=jnp.int32)
    zero = jnp.zeros((16,), jnp.int32)

    def chunk(i, accs):
        idx3 = (i * 16 + l16) * 3
        px = plsc.load_gather(xv, [idx3])
        py = plsc.load_gather(xv, [idx3 + 1])
        pz = plsc.load_gather(xv, [idx3 + 2])
        ev = _expert_id(px, py, pz)
        evv[pl.ds(i * 16, 16)] = ev
        return tuple(accs[e] + (ev == e).astype(jnp.int32) for e in range(E))

    accs = lax.fori_loop(0, C // 16, chunk, (zero,) * E)
    for e in range(E):
        cntv[pl.ds(e * 16, 16)] = accs[e]
    pltpu.sync_copy(evv, ev_hbm.at[w])
    pltpu.sync_copy(cntv, counts_hbm.at[w])


@functools.partial(
    pl.kernel,
    out_type=(
        jax.ShapeDtypeStruct((4 * NPAD,), jnp.float32),
        jax.ShapeDtypeStruct((N // 128, 128), jnp.int32),
        jax.ShapeDtypeStruct((G,), jnp.int32),
    ),
    mesh=_mesh,
    compiler_params=_params,
    scratch_types=(
        pltpu.VMEM((3 * C,), jnp.float32),
        pltpu.VMEM((C,), jnp.int32),
        pltpu.VMEM((NW, E * 16), jnp.int32),
        pltpu.VMEM((E, 16), jnp.int32),
        pltpu.VMEM((G,), jnp.int32),
        pltpu.VMEM((ROWS, 128), jnp.int32),
        pltpu.VMEM((4 * ROWS, 128), jnp.float32),
        pltpu.VMEM((4 * ROWS, 128), jnp.int32),
        pltpu.SemaphoreType.DMA,
    ),
)
def _sc_scatter(xflat_hbm, ev_hbm, counts_hbm, xs_hbm, d_hbm, b2e_hbm,
                xv, evv, cmv, base2, b2ev, dref, xcb, icb, sem):
    w = _wid()
    pltpu.sync_copy(xflat_hbm.at[pl.ds(w * 3 * C, 3 * C)], xv)
    pltpu.sync_copy(ev_hbm.at[w], evv)
    pltpu.sync_copy(counts_hbm, cmv)
    l16 = jnp.arange(16, dtype=jnp.int32)
    zero = jnp.zeros((16,), jnp.int32)
    ones = jnp.ones((16,), jnp.int32)
    onesf = jnp.ones((16,), jnp.float32)

    def waccum(wp, st):
        tots = list(st[:E])
        pres = list(st[E:])
        for e in range(E):
            row = cmv[wp, pl.ds(e * 16, 16)]
            tots[e] = tots[e] + row
            pres[e] = pres[e] + jnp.where(wp < w, row, 0)
        return tuple(tots) + tuple(pres)

    st = lax.fori_loop(0, NW, waccum, (zero,) * (2 * E))
    tot = zero
    for e in range(E):
        tot = tot + jnp.where(l16 == e, jnp.sum(st[e]), 0)
    cap = ((tot + (B - 1)) // B) * B
    padoff = plsc.cumsum(cap) - cap
    for e in range(E):
        own = cmv[w, pl.ds(e * 16, 16)]
        lanepre = plsc.cumsum(own) - own
        po_e = jnp.sum(jnp.where(l16 == e, padoff, 0))
        g_e = jnp.sum(st[E + e])
        base2[e, :] = lanepre + po_e + g_e

    @pl.when(w == 0)
    def _():
        ends = padoff + cap
        end_s = [jnp.sum(jnp.where(l16 == e, ends, 0)) for e in range(E)]
        for i in range(G // 16):
            gv = (i * 16 + l16) * B
            cntv = jnp.zeros((16,), jnp.int32)
            for e in range(E):
                cntv = cntv + (gv >= end_s[e]).astype(jnp.int32)
            b2ev[pl.ds(i * 16, 16)] = jnp.minimum(cntv, E - 1)
        pltpu.sync_copy(b2ev, b2e_hbm)

    def row_fn(r, _):
        for j in range(8):
            off = r * 128 + j * 16
            lane = j * 16 + l16
            ev = evv[pl.ds(off, 16)]
            d = plsc.load_gather(base2, [ev, l16])
            plsc.addupdate_scatter(base2, [ev, l16], ones)
            dref[r, pl.ds(j * 16, 16)] = d
            for c in range(3):
                v = plsc.load_gather(xv, [(off + l16) * 3 + c])
                plsc.store_scatter(xcb, [zero + (c * ROWS) + r, lane], v)
                plsc.store_scatter(icb, [zero + (c * ROWS) + r, lane], 4 * d + c)
            plsc.store_scatter(xcb, [zero + (3 * ROWS) + r, lane], onesf)
            plsc.store_scatter(icb, [zero + (3 * ROWS) + r, lane], 4 * d + 3)
        for c in range(4):
            pltpu.async_copy(
                xcb.at[c * ROWS + r], xs_hbm.at[icb.at[c * ROWS + r]], sem
            )
        return 0

    lax.fori_loop(0, ROWS, row_fn, 0)

    def drain(r, _):
        for c in range(4):
            pltpu.make_async_copy(
                xcb.at[c * ROWS + r], xs_hbm.at[icb.at[c * ROWS + r]], sem
            ).wait()
        return 0

    lax.fori_loop(0, ROWS, drain, 0)
    pltpu.sync_copy(dref, d_hbm.at[pl.ds(w * ROWS, ROWS), :])


@functools.partial(
    pl.kernel,
    out_type=jax.ShapeDtypeStruct((N // 128, 128), jnp.float32),
    mesh=_mesh,
    compiler_params=_params,
    scratch_types=(
        pltpu.VMEM((ROWS, 128), jnp.int32),
        pltpu.VMEM((ROWS, 128), jnp.float32),
        pltpu.SemaphoreType.DMA,
    ),
)
def _sc_gather_back(d_hbm, ys_hbm, y_hbm, dv, yv, sem):
    w = _wid()
    pltpu.sync_copy(d_hbm.at[pl.ds(w * ROWS, ROWS), :], dv)

    def fire(r, _):
        pltpu.async_copy(ys_hbm.at[dv.at[r]], yv.at[r], sem)
        return 0

    def drain(r, _):
        pltpu.make_async_copy(ys_hbm.at[dv.at[r]], yv.at[r], sem).wait()
        return 0

    lax.fori_loop(0, ROWS, fire, 0)
    lax.fori_loop(0, ROWS, drain, 0)
    pltpu.sync_copy(yv, y_hbm.at[pl.ds(w * ROWS, ROWS), :])


def _mlp_block(b2e_ref, x_ref, W1_ref, W2_ref, b2_ref, W3_ref, b3_ref, o_ref):
    h1 = jax.nn.relu(
        jnp.dot(x_ref[...], W1_ref[0], preferred_element_type=jnp.float32)
    )
    h2 = jax.nn.relu(
        jnp.dot(h1, W2_ref[0], preferred_element_type=jnp.float32) + b2_ref[0]
    )
    o_ref[...] = (
        jnp.dot(h2, W3_ref[0], preferred_element_type=jnp.float32) + b3_ref[0]
    )


def kernel(x, W1, b1, W2, b2, W3, b3):
    xflat = x.reshape(-1)
    counts, ev = _sc_route(xflat)
    xs_flat, d, b2e = _sc_scatter(xflat, ev, counts)
    xs = xs_flat.reshape(NPAD, 4)
    W1p = jnp.concatenate([W1, b1[:, None, :]], axis=1)
    b2r = b2.reshape(E, 1, H)
    b3r = b3.reshape(E, 1, 1)

    grid_spec = pltpu.PrefetchScalarGridSpec(
        num_scalar_prefetch=1,
        grid=(G,),
        in_specs=[
            pl.BlockSpec((B, 4), lambda g, b2e_ref: (g, 0)),
            pl.BlockSpec((1, 4, H), lambda g, b2e_ref: (b2e_ref[g], 0, 0)),
            pl.BlockSpec((1, H, H), lambda g, b2e_ref: (b2e_ref[g], 0, 0)),
            pl.BlockSpec((1, 1, H), lambda g, b2e_ref: (b2e_ref[g], 0, 0)),
            pl.BlockSpec((1, H, 1), lambda g, b2e_ref: (b2e_ref[g], 0, 0)),
            pl.BlockSpec((1, 1, 1), lambda g, b2e_ref: (b2e_ref[g], 0, 0)),
        ],
        out_specs=pl.BlockSpec((B, 1), lambda g, b2e_ref: (g, 0)),
    )
    ys = pl.pallas_call(
        _mlp_block,
        grid_spec=grid_spec,
        out_shape=jax.ShapeDtypeStruct((NPAD, 1), jnp.float32),
    )(b2e, xs, W1p, W2, b2r, W3, b3r)

    y = _sc_gather_back(d, ys.reshape(-1))
    return y.reshape(N, 1)


# speedup vs baseline: 2.4345x; 2.2115x over previous
import functools

import jax
import jax.numpy as jnp
from jax import lax
from jax.experimental import pallas as pl
from jax.experimental.pallas import tpu as pltpu
from jax.experimental.pallas import tpu_sc as plsc

E = 8
H = 128
N = 131072
NW = 32
C = N // NW
B = 1024
G = 176
NPAD = G * B
ROWS = C // 128
LROWS = ROWS + E

_mesh = plsc.VectorSubcoreMesh(core_axis_name="c", subcore_axis_name="s")
_params = pltpu.CompilerParams(needs_layout_passes=False)


def _wid():
    return lax.axis_index("s") * 2 + lax.axis_index("c")


def _expert_id(px, py, pz):
    c0 = ((px + 1.0) >= 1.0).astype(jnp.int32)
    c1 = ((py + 1.0) >= 1.0).astype(jnp.int32)
    c2 = ((pz + 1.0) >= 1.0).astype(jnp.int32)
    return c2 + 2 * c1 + 4 * c0


@functools.partial(
    pl.kernel,
    out_type=(
        jax.ShapeDtypeStruct((NW, E * 16), jnp.int32),
        jax.ShapeDtypeStruct((NW, 16), jnp.int32),
        jax.ShapeDtypeStruct((NW, C), jnp.int32),
    ),
    mesh=_mesh,
    compiler_params=_params,
    scratch_types=(
        pltpu.VMEM((3 * C,), jnp.float32),
        pltpu.VMEM((C,), jnp.int32),
        pltpu.VMEM((E * 16,), jnp.int32),
        pltpu.VMEM((16,), jnp.int32),
    ),
)
def _sc_route(xflat_hbm, counts_hbm, cnts_hbm, ev_hbm, xv, evv, cntv, csv):
    w = _wid()
    pltpu.sync_copy(xflat_hbm.at[pl.ds(w * 3 * C, 3 * C)], xv)
    l16 = jnp.arange(16, dtype=jnp.int32)
    zero = jnp.zeros((16,), jnp.int32)

    def chunk(i, accs):
        idx3 = (i * 16 + l16) * 3
        px = plsc.load_gather(xv, [idx3])
        py = plsc.load_gather(xv, [idx3 + 1])
        pz = plsc.load_gather(xv, [idx3 + 2])
        ev = _expert_id(px, py, pz)
        evv[pl.ds(i * 16, 16)] = ev
        return tuple(accs[e] + (ev == e).astype(jnp.int32) for e in range(E))

    accs = lax.fori_loop(0, C // 16, chunk, (zero,) * E)
    cs = zero
    for e in range(E):
        cntv[pl.ds(e * 16, 16)] = accs[e]
        cs = cs + jnp.where(l16 == e, jnp.sum(accs[e]), 0)
    csv[...] = cs
    pltpu.sync_copy(evv, ev_hbm.at[w])
    pltpu.sync_copy(cntv, counts_hbm.at[w])
    pltpu.sync_copy(csv, cnts_hbm.at[w])


@functools.partial(
    pl.kernel,
    out_type=(
        jax.ShapeDtypeStruct((4 * NPAD,), jnp.float32),
        jax.ShapeDtypeStruct((N // 128, 128), jnp.int32),
        jax.ShapeDtypeStruct((G,), jnp.int32),
    ),
    mesh=_mesh,
    compiler_params=_params,
    scratch_types=(
        pltpu.VMEM((3 * C,), jnp.float32),
        pltpu.VMEM((C,), jnp.int32),
        pltpu.VMEM((NW, E * 16), jnp.int32),
        pltpu.VMEM((NW, 16), jnp.int32),
        pltpu.VMEM((E, 16), jnp.int32),
        pltpu.VMEM((16,), jnp.int32),
        pltpu.VMEM((16,), jnp.int32),
        pltpu.VMEM((G,), jnp.int32),
        pltpu.VMEM((ROWS, 128), jnp.int32),
        pltpu.VMEM((4 * 128 * LROWS,), jnp.float32),
        pltpu.SemaphoreType.DMA,
    ),
)
def _sc_scatter(xflat_hbm, ev_hbm, counts_hbm, cnts_hbm, xs_hbm, d_hbm, b2e_hbm,
                xv, evv, cmv, csv, base2, loffv, goffv, b2ev, dref, xloc, sem):
    w = _wid()
    pltpu.sync_copy(xflat_hbm.at[pl.ds(w * 3 * C, 3 * C)], xv)
    pltpu.sync_copy(ev_hbm.at[w], evv)
    pltpu.sync_copy(counts_hbm, cmv)
    pltpu.sync_copy(cnts_hbm, csv)
    l16 = jnp.arange(16, dtype=jnp.int32)
    zero = jnp.zeros((16,), jnp.int32)
    ones = jnp.ones((16,), jnp.int32)
    onesf = jnp.ones((16,), jnp.float32)

    def waccum(wp, st):
        ecv, gprev = st
        row = csv[wp]
        r128 = ((row + 127) // 128) * 128
        return ecv + r128, gprev + jnp.where(wp < w, r128, 0)

    ecv, gprev = lax.fori_loop(0, NW, waccum, (zero, zero))
    ec_s = [jnp.sum(jnp.where(l16 == e, ecv, 0)) for e in range(E)]
    s_s = []
    run = jnp.int32(0)
    for e in range(E):
        s_s.append(run)
        run = run + ((ec_s[e] + (B - 1)) // B) * B
    sv = zero
    for e in range(E):
        sv = sv + jnp.where(l16 == e, s_s[e], 0)
    goffv[...] = sv + gprev
    own = csv[w]
    ownr = ((own + 127) // 128) * 128
    lpre = plsc.cumsum(ownr) - ownr
    loffv[...] = lpre
    for e in range(E):
        lane_cnt = cmv[w, pl.ds(e * 16, 16)]
        base2[e, :] = plsc.cumsum(lane_cnt) - lane_cnt

    @pl.when(w == 0)
    def _():
        end_s = []
        for e in range(E):
            end_s.append(s_s[e] + ((ec_s[e] + (B - 1)) // B) * B)
        for i in range(G // 16):
            gv = (i * 16 + l16) * B
            cntv = jnp.zeros((16,), jnp.int32)
            for e in range(E):
                cntv = cntv + (gv >= end_s[e]).astype(jnp.int32)
            b2ev[pl.ds(i * 16, 16)] = jnp.minimum(cntv, E - 1)
        pltpu.sync_copy(b2ev, b2e_hbm)

    def row_fn(r, _):
        for j in range(8):
            off = r * 128 + j * 16
            ev = evv[pl.ds(off, 16)]
            rank = plsc.load_gather(base2, [ev, l16])
            plsc.addupdate_scatter(base2, [ev, l16], ones)
            dloc = plsc.load_gather(loffv, [ev]) + rank
            dref[r, pl.ds(j * 16, 16)] = plsc.load_gather(goffv, [ev]) + rank
            for c in range(3):
                v = plsc.load_gather(xv, [(off + l16) * 3 + c])
                plsc.store_scatter(xloc, [4 * dloc + c], v)
            plsc.store_scatter(xloc, [4 * dloc + 3], onesf)
        return 0

    lax.fori_loop(0, ROWS, row_fn, 0)
    pltpu.sync_copy(dref, d_hbm.at[pl.ds(w * ROWS, ROWS), :])

    loc_s = [jnp.sum(jnp.where(l16 == e, lpre, 0)) for e in range(E)]
    go_s = [jnp.sum(jnp.where(l16 == e, sv + gprev, 0)) for e in range(E)]
    km_s = [jnp.sum(jnp.where(l16 == e, (own + 127) // 128, 0)) for e in range(E)]

    def fire(k, _):
        for e in range(E):
            @pl.when(k < km_s[e])
            def _():
                pltpu.async_copy(
                    xloc.at[pl.ds(pl.multiple_of((loc_s[e] + k * 128) * 4, 512), 512)],
                    xs_hbm.at[pl.ds(pl.multiple_of((go_s[e] + k * 128) * 4, 512), 512)],
                    sem,
                )
        return 0

    def drain(k, _):
        for e in range(E):
            @pl.when(k < km_s[e])
            def _():
                pltpu.make_async_copy(
                    xloc.at[pl.ds(pl.multiple_of((loc_s[e] + k * 128) * 4, 512), 512)],
                    xs_hbm.at[pl.ds(pl.multiple_of((go_s[e] + k * 128) * 4, 512), 512)],
                    sem,
                ).wait()
        return 0

    lax.fori_loop(0, ROWS, fire, 0)
    lax.fori_loop(0, ROWS, drain, 0)


@functools.partial(
    pl.kernel,
    out_type=jax.ShapeDtypeStruct((N // 128, 128), jnp.float32),
    mesh=_mesh,
    compiler_params=_params,
    scratch_types=(
        pltpu.VMEM((ROWS, 128), jnp.int32),
        pltpu.VMEM((ROWS, 128), jnp.float32),
        pltpu.SemaphoreType.DMA,
    ),
)
def _sc_gather_back(d_hbm, ys_hbm, y_hbm, dv, yv, sem):
    w = _wid()
    pltpu.sync_copy(d_hbm.at[pl.ds(w * ROWS, ROWS), :], dv)

    def fire(r, _):
        pltpu.async_copy(ys_hbm.at[dv.at[r]], yv.at[r], sem)
        return 0

    def drain(r, _):
        pltpu.make_async_copy(ys_hbm.at[dv.at[r]], yv.at[r], sem).wait()
        return 0

    lax.fori_loop(0, ROWS, fire, 0)
    lax.fori_loop(0, ROWS, drain, 0)
    pltpu.sync_copy(yv, y_hbm.at[pl.ds(w * ROWS, ROWS), :])


def _mlp_block(b2e_ref, x_ref, W1_ref, W2_ref, b2_ref, W3_ref, b3_ref, o_ref):
    h1 = jax.nn.relu(
        jnp.dot(x_ref[...], W1_ref[0], preferred_element_type=jnp.float32)
    )
    h2 = jax.nn.relu(
        jnp.dot(h1, W2_ref[0], preferred_element_type=jnp.float32) + b2_ref[0]
    )
    o_ref[...] = (
        jnp.dot(h2, W3_ref[0], preferred_element_type=jnp.float32) + b3_ref[0]
    )


def kernel(x, W1, b1, W2, b2, W3, b3):
    xflat = x.reshape(-1)
    counts, cnts, ev = _sc_route(xflat)
    xs_flat, d, b2e = _sc_scatter(xflat, ev, counts, cnts)
    xs = xs_flat.reshape(NPAD, 4)
    W1p = jnp.concatenate([W1, b1[:, None, :]], axis=1)
    b2r = b2.reshape(E, 1, H)
    b3r = b3.reshape(E, 1, 1)

    grid_spec = pltpu.PrefetchScalarGridSpec(
        num_scalar_prefetch=1,
        grid=(G,),
        in_specs=[
            pl.BlockSpec((B, 4), lambda g, b2e_ref: (g, 0)),
            pl.BlockSpec((1, 4, H), lambda g, b2e_ref: (b2e_ref[g], 0, 0)),
            pl.BlockSpec((1, H, H), lambda g, b2e_ref: (b2e_ref[g], 0, 0)),
            pl.BlockSpec((1, 1, H), lambda g, b2e_ref: (b2e_ref[g], 0, 0)),
            pl.BlockSpec((1, H, 1), lambda g, b2e_ref: (b2e_ref[g], 0, 0)),
            pl.BlockSpec((1, 1, 1), lambda g, b2e_ref: (b2e_ref[g], 0, 0)),
        ],
        out_specs=pl.BlockSpec((B, 1), lambda g, b2e_ref: (g, 0)),
    )
    ys = pl.pallas_call(
        _mlp_block,
        grid_spec=grid_spec,
        out_shape=jax.ShapeDtypeStruct((NPAD, 1), jnp.float32),
    )(b2e, xs, W1p, W2, b2r, W3, b3r)

    y = _sc_gather_back(d, ys.reshape(-1))
    return y.reshape(N, 1)


# speedup vs baseline: 4.1074x; 1.6872x over previous
import functools

import jax
import jax.numpy as jnp
from jax import lax
from jax.experimental import pallas as pl
from jax.experimental.pallas import tpu as pltpu
from jax.experimental.pallas import tpu_sc as plsc

E = 8
H = 128
N = 131072
NW = 32
C = N // NW
B = 1024
G = 176
NPAD = G * B
ROWS = C // 128
LGRP = ROWS + E
LW = 128 * LGRP

_mesh = plsc.VectorSubcoreMesh(core_axis_name="c", subcore_axis_name="s")
_params = pltpu.CompilerParams(needs_layout_passes=False)


def _wid():
    return lax.axis_index("s") * 2 + lax.axis_index("c")


def _expert_id(px, py, pz):
    c0 = ((px + 1.0) >= 1.0).astype(jnp.int32)
    c1 = ((py + 1.0) >= 1.0).astype(jnp.int32)
    c2 = ((pz + 1.0) >= 1.0).astype(jnp.int32)
    return c2 + 2 * c1 + 4 * c0


@functools.partial(
    pl.kernel,
    out_type=(
        jax.ShapeDtypeStruct((NW, E * 16), jnp.int32),
        jax.ShapeDtypeStruct((NW, 16), jnp.int32),
        jax.ShapeDtypeStruct((NW, C), jnp.int32),
    ),
    mesh=_mesh,
    compiler_params=_params,
    scratch_types=(
        pltpu.VMEM((3 * C,), jnp.float32),
        pltpu.VMEM((C,), jnp.int32),
        pltpu.VMEM((E * 16,), jnp.int32),
        pltpu.VMEM((16,), jnp.int32),
    ),
)
def _sc_route(xflat_hbm, counts_hbm, cnts_hbm, ev_hbm, xv, evv, cntv, csv):
    w = _wid()
    pltpu.sync_copy(xflat_hbm.at[pl.ds(w * 3 * C, 3 * C)], xv)
    l16 = jnp.arange(16, dtype=jnp.int32)
    zero = jnp.zeros((16,), jnp.int32)

    def chunk(i, accs):
        idx3 = (i * 16 + l16) * 3
        px = plsc.load_gather(xv, [idx3])
        py = plsc.load_gather(xv, [idx3 + 1])
        pz = plsc.load_gather(xv, [idx3 + 2])
        ev = _expert_id(px, py, pz)
        evv[pl.ds(i * 16, 16)] = ev
        return tuple(accs[e] + (ev == e).astype(jnp.int32) for e in range(E))

    accs = lax.fori_loop(0, C // 16, chunk, (zero,) * E)
    cs = zero
    for e in range(E):
        cntv[pl.ds(e * 16, 16)] = accs[e]
        cs = cs + jnp.where(l16 == e, jnp.sum(accs[e]), 0)
    csv[...] = cs
    pltpu.sync_copy(evv, ev_hbm.at[w])
    pltpu.sync_copy(cntv, counts_hbm.at[w])
    pltpu.sync_copy(csv, cnts_hbm.at[w])


@functools.partial(
    pl.kernel,
    out_type=(
        jax.ShapeDtypeStruct((4, NPAD), jnp.float32),
        jax.ShapeDtypeStruct((N // 128, 128), jnp.int32),
        jax.ShapeDtypeStruct((G,), jnp.int32),
    ),
    mesh=_mesh,
    compiler_params=_params,
    scratch_types=(
        pltpu.VMEM((3 * C,), jnp.float32),
        pltpu.VMEM((C,), jnp.int32),
        pltpu.VMEM((NW, E * 16), jnp.int32),
        pltpu.VMEM((NW, 16), jnp.int32),
        pltpu.VMEM((E, 16), jnp.int32),
        pltpu.VMEM((16,), jnp.int32),
        pltpu.VMEM((16,), jnp.int32),
        pltpu.VMEM((G,), jnp.int32),
        pltpu.VMEM((ROWS, 128), jnp.int32),
        pltpu.VMEM((4, LW), jnp.float32),
        pltpu.SemaphoreType.DMA,
    ),
)
def _sc_scatter(xflat_hbm, ev_hbm, counts_hbm, cnts_hbm, xs_hbm, d_hbm, b2e_hbm,
                xv, evv, cmv, csv, base2, loffv, goffv, b2ev, dref, xloc, sem):
    w = _wid()
    pltpu.sync_copy(xflat_hbm.at[pl.ds(w * 3 * C, 3 * C)], xv)
    pltpu.sync_copy(ev_hbm.at[w], evv)
    pltpu.sync_copy(counts_hbm, cmv)
    pltpu.sync_copy(cnts_hbm, csv)
    l16 = jnp.arange(16, dtype=jnp.int32)
    zero = jnp.zeros((16,), jnp.int32)
    ones = jnp.ones((16,), jnp.int32)
    onesf = jnp.ones((16,), jnp.float32)
    planes = [jnp.full((16,), c, jnp.int32) for c in range(4)]

    def waccum(wp, st):
        ecv, gprev = st
        row = csv[wp]
        r128 = ((row + 127) // 128) * 128
        return ecv + r128, gprev + jnp.where(wp < w, r128, 0)

    ecv, gprev = lax.fori_loop(0, NW, waccum, (zero, zero))
    ec_s = [jnp.sum(jnp.where(l16 == e, ecv, 0)) for e in range(E)]
    s_s = []
    run = jnp.int32(0)
    for e in range(E):
        s_s.append(run)
        run = run + ((ec_s[e] + (B - 1)) // B) * B
    sv = zero
    for e in range(E):
        sv = sv + jnp.where(l16 == e, s_s[e], 0)
    goffv[...] = sv + gprev
    own = csv[w]
    ownr = ((own + 127) // 128) * 128
    lpre = plsc.cumsum(ownr) - ownr
    loffv[...] = lpre
    for e in range(E):
        lane_cnt = cmv[w, pl.ds(e * 16, 16)]
        base2[e, :] = plsc.cumsum(lane_cnt) - lane_cnt

    @pl.when(w == 0)
    def _():
        end_s = []
        for e in range(E):
            end_s.append(s_s[e] + ((ec_s[e] + (B - 1)) // B) * B)
        for i in range(G // 16):
            gv = (i * 16 + l16) * B
            cntv = jnp.zeros((16,), jnp.int32)
            for e in range(E):
                cntv = cntv + (gv >= end_s[e]).astype(jnp.int32)
            b2ev[pl.ds(i * 16, 16)] = jnp.minimum(cntv, E - 1)
        pltpu.sync_copy(b2ev, b2e_hbm)

    def row_fn(r, _):
        for j in range(8):
            off = r * 128 + j * 16
            ev = evv[pl.ds(off, 16)]
            rank = plsc.load_gather(base2, [ev, l16])
            plsc.addupdate_scatter(base2, [ev, l16], ones)
            dloc = plsc.load_gather(loffv, [ev]) + rank
            dglob = plsc.load_gather(goffv, [ev]) + rank
            dref[r, pl.ds(j * 16, 16)] = dglob
            for c in range(3):
                v = plsc.load_gather(xv, [(off + l16) * 3 + c])
                plsc.store_scatter(xloc, [planes[c], dloc], v)
            plsc.store_scatter(xloc, [planes[3], dloc], onesf)
        return 0

    lax.fori_loop(0, ROWS, row_fn, 0)
    pltpu.sync_copy(dref, d_hbm.at[pl.ds(w * ROWS, ROWS), :])

    loc_s = [jnp.sum(jnp.where(l16 == e, lpre, 0)) for e in range(E)]
    go_s = [jnp.sum(jnp.where(l16 == e, sv + gprev, 0)) for e in range(E)]
    km_s = [jnp.sum(jnp.where(l16 == e, (own + 127) // 128, 0)) for e in range(E)]

    def fire(k, _):
        for e in range(E):
            @pl.when(k < km_s[e])
            def _():
                for c in range(4):
                    pltpu.async_copy(
                        xloc.at[c, pl.ds(pl.multiple_of(loc_s[e] + k * 128, 128), 128)],
                        xs_hbm.at[c, pl.ds(pl.multiple_of(go_s[e] + k * 128, 128), 128)],
                        sem,
                    )
        return 0

    def drain(k, _):
        for e in range(E):
            @pl.when(k < km_s[e])
            def _():
                for c in range(4):
                    pltpu.make_async_copy(
                        xloc.at[c, pl.ds(pl.multiple_of(loc_s[e] + k * 128, 128), 128)],
                        xs_hbm.at[c, pl.ds(pl.multiple_of(go_s[e] + k * 128, 128), 128)],
                        sem,
                    ).wait()
        return 0

    lax.fori_loop(0, ROWS, fire, 0)
    lax.fori_loop(0, ROWS, drain, 0)


@functools.partial(
    pl.kernel,
    out_type=jax.ShapeDtypeStruct((N // 128, 128), jnp.float32),
    mesh=_mesh,
    compiler_params=_params,
    scratch_types=(
        pltpu.VMEM((ROWS, 128), jnp.int32),
        pltpu.VMEM((ROWS, 128), jnp.float32),
        pltpu.SemaphoreType.DMA,
    ),
)
def _sc_gather_back(d_hbm, ys_hbm, y_hbm, dv, yv, sem):
    w = _wid()
    pltpu.sync_copy(d_hbm.at[pl.ds(w * ROWS, ROWS), :], dv)

    def fire(r, _):
        pltpu.async_copy(ys_hbm.at[dv.at[r]], yv.at[r], sem)
        return 0

    def drain(r, _):
        pltpu.make_async_copy(ys_hbm.at[dv.at[r]], yv.at[r], sem).wait()
        return 0

    lax.fori_loop(0, ROWS, fire, 0)
    lax.fori_loop(0, ROWS, drain, 0)
    pltpu.sync_copy(yv, y_hbm.at[pl.ds(w * ROWS, ROWS), :])


def _mlp_block(b2e_ref, x_ref, W1_ref, W2_ref, b2_ref, W3_ref, b3_ref, o_ref):
    xp = x_ref[...]
    h1 = jax.nn.relu(
        jnp.dot(W1_ref[0], xp, preferred_element_type=jnp.float32)
    )
    h2 = jax.nn.relu(
        jnp.dot(W2_ref[0], h1, preferred_element_type=jnp.float32) + b2_ref[0]
    )
    yt = jnp.dot(W3_ref[0], h2, preferred_element_type=jnp.float32) + b3_ref[0]
    o_ref[...] = yt


def kernel(x, W1, b1, W2, b2, W3, b3):
    xflat = x.reshape(-1)
    counts, cnts, ev = _sc_route(xflat)
    xs, d, b2e = _sc_scatter(xflat, ev, counts, cnts)
    W1p = jnp.concatenate([W1, b1[:, None, :]], axis=1)
    W1T = W1p.transpose(0, 2, 1)
    W2T = W2.transpose(0, 2, 1)
    b2c = b2[:, :, None]
    W3T = W3.transpose(0, 2, 1)
    b3r = b3.reshape(E, 1, 1)

    grid_spec = pltpu.PrefetchScalarGridSpec(
        num_scalar_prefetch=1,
        grid=(G,),
        in_specs=[
            pl.BlockSpec((4, B), lambda g, b2e_ref: (0, g)),
            pl.BlockSpec((1, H, 4), lambda g, b2e_ref: (b2e_ref[g], 0, 0)),
            pl.BlockSpec((1, H, H), lambda g, b2e_ref: (b2e_ref[g], 0, 0)),
            pl.BlockSpec((1, H, 1), lambda g, b2e_ref: (b2e_ref[g], 0, 0)),
            pl.BlockSpec((1, 1, H), lambda g, b2e_ref: (b2e_ref[g], 0, 0)),
            pl.BlockSpec((1, 1, 1), lambda g, b2e_ref: (b2e_ref[g], 0, 0)),
        ],
        out_specs=pl.BlockSpec((1, B), lambda g, b2e_ref: (0, g)),
    )
    ys = pl.pallas_call(
        _mlp_block,
        grid_spec=grid_spec,
        out_shape=jax.ShapeDtypeStruct((1, NPAD), jnp.float32),
    )(b2e, xs, W1T, W2T, b2c, W3T, b3r)

    y = _sc_gather_back(d, ys.reshape(-1))
    return y.reshape(N, 1)
